# baseline (speedup 1.0000x reference)
import functools

import jax
import jax.numpy as jnp
from jax import lax
from jax.experimental import pallas as pl
from jax.experimental.pallas import tpu as pltpu
from jax.experimental.pallas import tpu_sc as plsc

NE = 320000
NN = 10000
D = 128
HD = D // 2
CW = 16
WD = 64 + 16
NC = 2
NS = 16
EPT = NE // NS
B = 80
NB = EPT // B
RPT = 624
TAIL = NN - NS * RPT
ZR = 48
NBUF = 5
LOOK = 3

_mesh = plsc.VectorSubcoreMesh(
    core_axis_name="c", subcore_axis_name="s", num_cores=NC, num_subcores=NS
)


@functools.partial(
    pl.kernel,
    out_type=jax.ShapeDtypeStruct((NN, D), jnp.float32),
    mesh=_mesh,
    compiler_params=pltpu.CompilerParams(use_tc_tiling_on_sc=False),
    scratch_types=[
        pltpu.VMEM((NB, B), jnp.int32),
        pltpu.VMEM((NB, B), jnp.int32),
        pltpu.VMEM((NBUF, B, WD), jnp.float32),
        pltpu.VMEM((ZR, WD), jnp.float32),
        pltpu.VMEM_SHARED((NN, WD), jnp.float32),
        pltpu.SemaphoreType.DMA((NBUF,)),
        pltpu.SemaphoreType.DMA((NBUF,)),
    ],
)
def _sc_gather_scatter(
    et_hbm, dst_hbm, tlo_hbm, thi_hbm, out_hbm,
    et_v, dst_v, rows_v, zbuf, acc_sh,
    sem_g, sem_s,
):
    c = lax.axis_index("c")
    s = lax.axis_index("s")

    pltpu.sync_copy(et_hbm.at[s], et_v)
    pltpu.sync_copy(dst_hbm.at[s], dst_v)

    z16 = jnp.zeros((16,), jnp.float32)

    def _zrow(r, carry):
        for k in range(WD // 16):
            zbuf[r, pl.ds(k * 16, 16)] = z16
        return carry

    lax.fori_loop(0, ZR, _zrow, None)

    for t in range(RPT // ZR):
        pltpu.sync_copy(zbuf, acc_sh.at[pl.ds(s * RPT + t * ZR, ZR)])

    @pl.when(s == NS - 1)
    def _zero_tail():
        pltpu.sync_copy(zbuf.at[pl.ds(0, TAIL)], acc_sh.at[pl.ds(NS * RPT, TAIL)])

    plsc.subcore_barrier()

    def _gather_start(j, b):
        @pl.when(c == 0)
        def _g0():
            pltpu.async_copy(tlo_hbm.at[et_v.at[j]], rows_v.at[b], sem_g.at[b])

        @pl.when(c == 1)
        def _g1():
            pltpu.async_copy(thi_hbm.at[et_v.at[j]], rows_v.at[b], sem_g.at[b])

    def _gather_wait(b):
        pltpu.make_async_copy(
            tlo_hbm.at[et_v.at[0]], rows_v.at[b], sem_g.at[b]
        ).wait()

    def _scatter_wait(b):
        pltpu.make_async_copy(
            rows_v.at[b], acc_sh.at[dst_v.at[0]], sem_s.at[b]
        ).wait()

    for b in range(LOOK):
        _gather_start(b, b)

    def _group(g, carry):
        for b in range(NBUF):
            j = g * NBUF + b
            b5 = (b + LOOK) % NBUF
            _gather_wait(b)
            pltpu.async_copy(rows_v.at[b], acc_sh.at[dst_v.at[j]], sem_s.at[b], add=True)

            @pl.when(j >= NBUF - LOOK)
            def _drain_scatter():
                _scatter_wait(b5)

            @pl.when(j + LOOK < NB)
            def _next_gather():
                _gather_start(j + LOOK, b5)

        return carry

    lax.fori_loop(0, NB // NBUF, _group, None)

    for b in range(LOOK, NBUF):
        _scatter_wait(b)

    plsc.subcore_barrier()

    def _finish_rows(row0, nrows):
        pltpu.sync_copy(acc_sh.at[pl.ds(row0, nrows)], zbuf.at[pl.ds(0, nrows)])

        def _frow(r, carry):
            cnt16 = zbuf[r, pl.ds(HD, CW)]
            recip = jnp.where(cnt16 > 0.0, 1.0 / jnp.maximum(cnt16, 1.0), 0.0)
            for k in range(HD // 16):
                v = zbuf[r, pl.ds(k * 16, 16)]
                zbuf[r, pl.ds(k * 16, 16)] = v * recip
            return carry

        lax.fori_loop(0, nrows, _frow, None)
        pltpu.sync_copy(
            zbuf.at[pl.ds(0, nrows), pl.ds(0, HD)],
            out_hbm.at[pl.ds(row0, nrows), pl.ds(c * HD, HD)],
        )

    def _fin_chunk(t, carry):
        _finish_rows(s * RPT + t * ZR, ZR)
        return carry

    lax.fori_loop(0, RPT // ZR, _fin_chunk, None)

    @pl.when(s == NS - 1)
    def _fin_tail():
        _finish_rows(NS * RPT, TAIL)


def kernel(etypes, dst, num_nodes, rel_head_emb, rel_tail_emb):
    num_rel = rel_head_emb.shape[0]
    table = jnp.concatenate([rel_head_emb, rel_tail_emb], axis=0)
    ones_cols = jnp.ones((2 * num_rel, CW), jnp.float32)
    tlo = jnp.concatenate([table[:, :HD], ones_cols], axis=1)
    thi = jnp.concatenate([table[:, HD:], ones_cols], axis=1)
    del num_rel
    et = etypes.reshape(NS, NB, B)
    ds_ = dst.reshape(NS, NB, B)
    return _sc_gather_scatter(et, ds_, tlo, thi)
